# baseline (speedup 1.0000x reference)
import jax
import jax.numpy as jnp
from jax import lax
from jax.experimental import pallas as pl
from jax.experimental.pallas import tpu as pltpu
from jax.experimental.pallas import tpu_sc as plsc

_N = 10000
_D = 128
_DH = _D // 2
_E = 320000
_NC = 2
_NS = 16
_NW = _NC * _NS
_TPW = _E // _NW
_C = 64
_NCH = 160
_EPAD = _NW * _NCH * _C
_NPAD = _N + 16
_NBUF = 3
_RPZ = 626
_RPD = 625
_RB = 2000
_GRID = _N // _RB
_EPS = 1e-5


def _mm_m_body(h_ref, wn_ref, m_ref):
    m_ref[...] = jnp.dot(h_ref[...], wn_ref[...],
                         preferred_element_type=jnp.float32)


def _mm_m(h, wn):
    return pl.pallas_call(
        _mm_m_body,
        grid=(_GRID,),
        in_specs=[
            pl.BlockSpec((_RB, _D), lambda i: (i, 0)),
            pl.BlockSpec((_D, _D), lambda i: (0, 0)),
        ],
        out_specs=pl.BlockSpec((_RB, _D), lambda i: (i, 0)),
        out_shape=jax.ShapeDtypeStruct((_N, _D), jnp.float32),
    )(h, wn)


def _bnr_m_body(pre_ref, stats_ref, gamma_ref, beta_ref, wn_ref,
                m_ref, h_ref):
    stats = stats_ref[...]
    mean = stats[0:1, :] * (1.0 / _N)
    var = stats[1:2, :] * (1.0 / _N) - mean * mean
    rstd = lax.rsqrt(var + _EPS)
    h = (pre_ref[...] - mean) * (rstd * gamma_ref[...]) + beta_ref[...]
    h = jnp.maximum(h, 0.0)
    h_ref[...] = h
    m_ref[...] = jnp.dot(h, wn_ref[...], preferred_element_type=jnp.float32)


def _bnr_m(pre, stats, gamma, beta, wn):
    return pl.pallas_call(
        _bnr_m_body,
        grid=(_GRID,),
        in_specs=[
            pl.BlockSpec((_RB, _D), lambda i: (i, 0)),
            pl.BlockSpec((2, _D), lambda i: (0, 0)),
            pl.BlockSpec((1, _D), lambda i: (0, 0)),
            pl.BlockSpec((1, _D), lambda i: (0, 0)),
            pl.BlockSpec((_D, _D), lambda i: (0, 0)),
        ],
        out_specs=[
            pl.BlockSpec((_RB, _D), lambda i: (i, 0)),
            pl.BlockSpec((_RB, _D), lambda i: (i, 0)),
        ],
        out_shape=[
            jax.ShapeDtypeStruct((_N, _D), jnp.float32),
            jax.ShapeDtypeStruct((_N, _D), jnp.float32),
        ],
    )(pre, stats, gamma, beta, wn)


def _pre_from_parts(h_ref, ws_ref, agg_ref, degp_ref, b_ref):
    deg = degp_ref[0, :, 0] + degp_ref[1, :, 0]
    rdeg = 1.0 / jnp.maximum(deg, 1.0)
    agg = agg_ref[0] + agg_ref[1]
    s = jnp.dot(h_ref[...], ws_ref[...], preferred_element_type=jnp.float32)
    return s + agg * rdeg[:, None] + b_ref[...]


def _post_body(h_ref, ws_ref, agg_ref, degp_ref, b_ref, pre_ref, stats_ref):
    pre = _pre_from_parts(h_ref, ws_ref, agg_ref, degp_ref, b_ref)
    pre_ref[...] = pre

    @pl.when(pl.program_id(0) == 0)
    def _():
        stats_ref[...] = jnp.zeros_like(stats_ref)

    blk = jnp.concatenate(
        [jnp.sum(pre, axis=0)[None], jnp.sum(pre * pre, axis=0)[None]], axis=0)
    stats_ref[...] += blk


def _post(h, ws, aggp, degp, b):
    return pl.pallas_call(
        _post_body,
        grid=(_GRID,),
        in_specs=[
            pl.BlockSpec((_RB, _D), lambda i: (i, 0)),
            pl.BlockSpec((_D, _D), lambda i: (0, 0)),
            pl.BlockSpec((2, _RB, _D), lambda i: (0, i, 0)),
            pl.BlockSpec((2, _RB, 16), lambda i: (0, i, 0)),
            pl.BlockSpec((1, _D), lambda i: (0, 0)),
        ],
        out_specs=[
            pl.BlockSpec((_RB, _D), lambda i: (i, 0)),
            pl.BlockSpec((2, _D), lambda i: (0, 0)),
        ],
        out_shape=[
            jax.ShapeDtypeStruct((_N, _D), jnp.float32),
            jax.ShapeDtypeStruct((2, _D), jnp.float32),
        ],
    )(h, ws, aggp, degp, b)


def _post_lsm_body(h_ref, ws_ref, agg_ref, degp_ref, b_ref, out_ref):
    pre = _pre_from_parts(h_ref, ws_ref, agg_ref, degp_ref, b_ref)
    mx = jnp.max(pre, axis=1, keepdims=True)
    ex = jnp.exp(pre - mx)
    lse = jnp.log(jnp.sum(ex, axis=1, keepdims=True)) + mx
    out_ref[...] = pre - lse


def _post_lsm(h, ws, aggp, degp, b):
    return pl.pallas_call(
        _post_lsm_body,
        grid=(_GRID,),
        in_specs=[
            pl.BlockSpec((_RB, _D), lambda i: (i, 0)),
            pl.BlockSpec((_D, _D), lambda i: (0, 0)),
            pl.BlockSpec((2, _RB, _D), lambda i: (0, i, 0)),
            pl.BlockSpec((2, _RB, 16), lambda i: (0, i, 0)),
            pl.BlockSpec((1, _D), lambda i: (0, 0)),
        ],
        out_specs=pl.BlockSpec((_RB, _D), lambda i: (i, 0)),
        out_shape=jax.ShapeDtypeStruct((_N, _D), jnp.float32),
    )(h, ws, aggp, degp, b)


def _sc_agg_call(m, srcm, dstm, zeros_nd):
    mesh = plsc.VectorSubcoreMesh(core_axis_name="c", subcore_axis_name="s")

    out_type = [jax.ShapeDtypeStruct((_NC, _N, _D), jnp.float32)]
    scratch = [
        pltpu.VMEM((_NCH, _C), jnp.int32),
        pltpu.VMEM((_NCH, _C), jnp.int32),
        [pltpu.VMEM((_C, _D), jnp.float32) for _ in range(_NBUF)],
        [pltpu.SemaphoreType.DMA for _ in range(_NBUF)],
        [pltpu.SemaphoreType.DMA for _ in range(_NBUF)],
        pltpu.VMEM_SHARED((_NPAD, _D), jnp.float32),
    ]

    def body(m_hbm, srcm_hbm, dstm_hbm, z_nd_hbm, out_agg,
             src_v, dst_v, bufs, gsems, ssems, agg_sh):
        cid = lax.axis_index("c")
        sid = lax.axis_index("s")
        wid = cid * _NS + sid

        pltpu.sync_copy(z_nd_hbm.at[pl.ds(sid * _RPZ, _RPZ)],
                        agg_sh.at[pl.ds(sid * _RPZ, _RPZ)])
        pltpu.sync_copy(srcm_hbm.at[pl.ds(wid * _NCH, _NCH)], src_v)
        pltpu.sync_copy(dstm_hbm.at[pl.ds(wid * _NCH, _NCH)], dst_v)
        plsc.subcore_barrier()

        def gather(g, b):
            pltpu.async_copy(m_hbm.at[src_v.at[g]], bufs[b], gsems[b])

        def gwait(b):
            pltpu.make_async_copy(m_hbm.at[src_v.at[0]], bufs[b],
                                  gsems[b]).wait()

        def scat(g, b):
            pltpu.async_copy(bufs[b], agg_sh.at[dst_v.at[g]],
                             ssems[b], add=True)

        def swait(b):
            pltpu.make_async_copy(bufs[b], agg_sh.at[dst_v.at[0]],
                                  ssems[b]).wait()

        gather(0, 0)
        gather(1, 1)
        gwait(0)
        scat(0, 0)
        gather(2, 2)

        def super_iter(i, carry):
            for j in range(_NBUF):
                g = _NBUF * i + 1 + j
                b = (1 + j) % _NBUF
                gwait(b)
                scat(g, b)
                nb = (b + 2) % _NBUF
                swait(nb)
                gather(g + 2, nb)
            return carry

        lax.fori_loop(0, (_NCH - 4) // _NBUF, super_iter, 0)

        g = _NCH - 3
        gwait(g % _NBUF)
        scat(g, g % _NBUF)
        swait((g + 2) % _NBUF)
        gather(g + 2, (g + 2) % _NBUF)
        for g in (_NCH - 2, _NCH - 1):
            gwait(g % _NBUF)
            scat(g, g % _NBUF)
        for g in (_NCH - 3, _NCH - 2, _NCH - 1):
            swait(g % _NBUF)

        plsc.subcore_barrier()

        pltpu.sync_copy(agg_sh.at[pl.ds(sid * _RPD, _RPD)],
                        out_agg.at[cid, pl.ds(sid * _RPD, _RPD)])

    return pl.kernel(body, out_type=out_type, mesh=mesh,
                     scratch_types=scratch,
                     compiler_params=pltpu.CompilerParams(
                         use_tc_tiling_on_sc=False))(
        m, srcm, dstm, zeros_nd)[0]


def _sc_deg_call(dstm, zeros16, ones16):
    mesh = plsc.VectorSubcoreMesh(core_axis_name="c", subcore_axis_name="s")
    out_type = [jax.ShapeDtypeStruct((_NC, _N, 16), jnp.float32)]
    scratch = [
        pltpu.VMEM((_NCH, _C), jnp.int32),
        pltpu.VMEM((_C, 16), jnp.float32),
        pltpu.VMEM_SHARED((_NPAD, 16), jnp.float32),
        pltpu.SemaphoreType.DMA,
        pltpu.SemaphoreType.DMA,
    ]

    def body(dstm_hbm, z16_hbm, ones_hbm, out_deg,
             dst_v, ones_v, deg_sh, sem_a, sem_b):
        cid = lax.axis_index("c")
        sid = lax.axis_index("s")
        wid = cid * _NS + sid

        pltpu.sync_copy(z16_hbm.at[pl.ds(sid * _RPZ, _RPZ)],
                        deg_sh.at[pl.ds(sid * _RPZ, _RPZ)])
        pltpu.sync_copy(ones_hbm, ones_v)
        pltpu.sync_copy(dstm_hbm.at[pl.ds(wid * _NCH, _NCH)], dst_v)
        plsc.subcore_barrier()

        def scat(g, sem):
            pltpu.async_copy(ones_v, deg_sh.at[dst_v.at[g]], sem, add=True)

        def swait(sem):
            pltpu.make_async_copy(ones_v, deg_sh.at[dst_v.at[0]], sem).wait()

        scat(0, sem_a)
        scat(1, sem_b)

        def step(i, carry):
            g = 2 * i

            @pl.when(g + 2 < _NCH)
            def _():
                scat(g + 2, sem_a)
                scat(g + 3, sem_b)

            swait(sem_a)
            swait(sem_b)
            return carry

        lax.fori_loop(0, _NCH // 2, step, 0)
        plsc.subcore_barrier()

        pltpu.sync_copy(deg_sh.at[pl.ds(sid * _RPD, _RPD)],
                        out_deg.at[cid, pl.ds(sid * _RPD, _RPD)])

    return pl.kernel(body, out_type=out_type, mesh=mesh,
                     scratch_types=scratch,
                     compiler_params=pltpu.CompilerParams(
                         use_tc_tiling_on_sc=False))(
        dstm, zeros16, ones16)[0]


def kernel(x, edge_index, W_self0, W_neigh0, b0, gamma0, beta0,
           W_self1, W_neigh1, b1, gamma1, beta1, W_self2, W_neigh2, b2):
    npad = _EPAD - _E
    srcm = jnp.concatenate(
        [edge_index[0], jnp.zeros((npad,), jnp.int32)]).reshape(
            _EPAD // _C, _C)
    trash = _N + (jnp.arange(npad, dtype=jnp.int32) % 16)
    dstm = jnp.concatenate([edge_index[1], trash]).reshape(_EPAD // _C, _C)
    zeros_nd = jnp.zeros((_NPAD, _D), jnp.float32)
    zeros16 = jnp.zeros((_NPAD, 16), jnp.float32)
    ones16 = jnp.ones((_C, 16), jnp.float32)
    b0r, b1r, b2r = (b.reshape(1, _D) for b in (b0, b1, b2))
    g0r, g1r = gamma0.reshape(1, _D), gamma1.reshape(1, _D)
    be0r, be1r = beta0.reshape(1, _D), beta1.reshape(1, _D)

    degp = _sc_deg_call(dstm, zeros16, ones16)
    m0 = _mm_m(x, W_neigh0)
    aggp0 = _sc_agg_call(m0, srcm, dstm, zeros_nd)
    pre1, stats1 = _post(x, W_self0, aggp0, degp, b0r)
    m1, h1 = _bnr_m(pre1, stats1, g0r, be0r, W_neigh1)
    aggp1 = _sc_agg_call(m1, srcm, dstm, zeros_nd)
    pre2, stats2 = _post(h1, W_self1, aggp1, degp, b1r)
    m2, h2 = _bnr_m(pre2, stats2, g1r, be1r, W_neigh2)
    aggp2 = _sc_agg_call(m2, srcm, dstm, zeros_nd)
    return _post_lsm(h2, W_self2, aggp2, degp, b2r)


# speedup vs baseline: 3.7795x; 3.7795x over previous
import jax
import jax.numpy as jnp
from jax import lax
from jax.experimental import pallas as pl
from jax.experimental.pallas import tpu as pltpu
from jax.experimental.pallas import tpu_sc as plsc

_N = 10000
_D = 128
_DH = _D // 2
_E = 320000
_NC = 2
_NS = 16
_NW = _NC * _NS
_TPW = _E // _NW
_C = 125
_NCH = _TPW // _C
_NBUF = 8
_PF = 4
_ZW = 10
_RPZ = _N // _ZW
_RB = 2000
_GRID = _N // _RB
_EPS = 1e-5


def _mm_m_body(h_ref, wn_ref, m_ref):
    m = jnp.dot(h_ref[...], wn_ref[...], preferred_element_type=jnp.float32)
    m_ref[0] = m[:, :_DH]
    m_ref[1] = m[:, _DH:]


def _mm_m(h, wn):
    return pl.pallas_call(
        _mm_m_body,
        grid=(_GRID,),
        in_specs=[
            pl.BlockSpec((_RB, _D), lambda i: (i, 0)),
            pl.BlockSpec((_D, _D), lambda i: (0, 0)),
        ],
        out_specs=pl.BlockSpec((2, _RB, _DH), lambda i: (0, i, 0)),
        out_shape=jax.ShapeDtypeStruct((2, _N, _DH), jnp.float32),
    )(h, wn)


def _bnr_m_body(pre_ref, stats_ref, gamma_ref, beta_ref, wn_ref,
                m_ref, h_ref):
    stats = stats_ref[...]
    mean = stats[0:1, :] * (1.0 / _N)
    var = stats[1:2, :] * (1.0 / _N) - mean * mean
    rstd = lax.rsqrt(var + _EPS)
    h = (pre_ref[...] - mean) * (rstd * gamma_ref[...]) + beta_ref[...]
    h = jnp.maximum(h, 0.0)
    h_ref[...] = h
    m = jnp.dot(h, wn_ref[...], preferred_element_type=jnp.float32)
    m_ref[0] = m[:, :_DH]
    m_ref[1] = m[:, _DH:]


def _bnr_m(pre, stats, gamma, beta, wn):
    return pl.pallas_call(
        _bnr_m_body,
        grid=(_GRID,),
        in_specs=[
            pl.BlockSpec((_RB, _D), lambda i: (i, 0)),
            pl.BlockSpec((2, _D), lambda i: (0, 0)),
            pl.BlockSpec((1, _D), lambda i: (0, 0)),
            pl.BlockSpec((1, _D), lambda i: (0, 0)),
            pl.BlockSpec((_D, _D), lambda i: (0, 0)),
        ],
        out_specs=[
            pl.BlockSpec((2, _RB, _DH), lambda i: (0, i, 0)),
            pl.BlockSpec((_RB, _D), lambda i: (i, 0)),
        ],
        out_shape=[
            jax.ShapeDtypeStruct((2, _N, _DH), jnp.float32),
            jax.ShapeDtypeStruct((_N, _D), jnp.float32),
        ],
    )(pre, stats, gamma, beta, wn)


def _post_bnr_body(h_ref, ws_ref, agg_ref, degp_ref, b_ref,
                   gamma_ref, beta_ref, wn_ref,
                   m_ref, hn_ref, pre_scr, stats_scr):
    p = pl.program_id(0)
    i = pl.program_id(1)

    @pl.when(p == 0)
    def _():
        pre = _pre_from_parts(h_ref, ws_ref, agg_ref, degp_ref, b_ref)
        pre_scr[pl.ds(i * _RB, _RB), :] = pre

        @pl.when(i == 0)
        def _():
            stats_scr[...] = jnp.zeros_like(stats_scr)

        blk = jnp.concatenate(
            [jnp.sum(pre, axis=0)[None], jnp.sum(pre * pre, axis=0)[None]],
            axis=0)
        stats_scr[...] += blk

    @pl.when(p == 1)
    def _():
        stats = stats_scr[...]
        mean = stats[0:1, :] * (1.0 / _N)
        var = stats[1:2, :] * (1.0 / _N) - mean * mean
        rstd = lax.rsqrt(var + _EPS)
        pre = pre_scr[pl.ds(i * _RB, _RB), :]
        h = (pre - mean) * (rstd * gamma_ref[...]) + beta_ref[...]
        h = jnp.maximum(h, 0.0)
        hn_ref[...] = h
        m = jnp.dot(h, wn_ref[...], preferred_element_type=jnp.float32)
        m_ref[0] = m[:, :_DH]
        m_ref[1] = m[:, _DH:]


def _post_bnr(h, ws, aggp, degp, b, gamma, beta, wn):
    return pl.pallas_call(
        _post_bnr_body,
        grid=(2, _GRID),
        in_specs=[
            pl.BlockSpec((_RB, _D), lambda p, i: ((1 - p) * i, 0)),
            pl.BlockSpec((_D, _D), lambda p, i: (0, 0)),
            pl.BlockSpec((2, _RB, _D), lambda p, i: (0, (1 - p) * i, 0)),
            pl.BlockSpec((2, _RB, 16), lambda p, i: (0, (1 - p) * i, 0)),
            pl.BlockSpec((1, _D), lambda p, i: (0, 0)),
            pl.BlockSpec((1, _D), lambda p, i: (0, 0)),
            pl.BlockSpec((1, _D), lambda p, i: (0, 0)),
            pl.BlockSpec((_D, _D), lambda p, i: (0, 0)),
        ],
        out_specs=[
            pl.BlockSpec((2, _RB, _DH), lambda p, i: (0, p * i, 0)),
            pl.BlockSpec((_RB, _D), lambda p, i: (p * i, 0)),
        ],
        out_shape=[
            jax.ShapeDtypeStruct((2, _N, _DH), jnp.float32),
            jax.ShapeDtypeStruct((_N, _D), jnp.float32),
        ],
        scratch_shapes=[
            pltpu.VMEM((_N, _D), jnp.float32),
            pltpu.VMEM((2, _D), jnp.float32),
        ],
    )(h, ws, aggp, degp, b, gamma, beta, wn)


def _pre_from_parts(h_ref, ws_ref, agg_ref, degp_ref, b_ref):
    deg = degp_ref[0, :, 0] + degp_ref[1, :, 0]
    rdeg = 1.0 / jnp.maximum(deg, 1.0)
    agg = agg_ref[0] + agg_ref[1]
    s = jnp.dot(h_ref[...], ws_ref[...], preferred_element_type=jnp.float32)
    return s + agg * rdeg[:, None] + b_ref[...]


def _post_body(h_ref, ws_ref, agg_ref, degp_ref, b_ref, pre_ref, stats_ref):
    pre = _pre_from_parts(h_ref, ws_ref, agg_ref, degp_ref, b_ref)
    pre_ref[...] = pre

    @pl.when(pl.program_id(0) == 0)
    def _():
        stats_ref[...] = jnp.zeros_like(stats_ref)

    blk = jnp.concatenate(
        [jnp.sum(pre, axis=0)[None], jnp.sum(pre * pre, axis=0)[None]], axis=0)
    stats_ref[...] += blk


def _post(h, ws, aggp, degp, b):
    return pl.pallas_call(
        _post_body,
        grid=(_GRID,),
        in_specs=[
            pl.BlockSpec((_RB, _D), lambda i: (i, 0)),
            pl.BlockSpec((_D, _D), lambda i: (0, 0)),
            pl.BlockSpec((2, _RB, _D), lambda i: (0, i, 0)),
            pl.BlockSpec((2, _RB, 16), lambda i: (0, i, 0)),
            pl.BlockSpec((1, _D), lambda i: (0, 0)),
        ],
        out_specs=[
            pl.BlockSpec((_RB, _D), lambda i: (i, 0)),
            pl.BlockSpec((2, _D), lambda i: (0, 0)),
        ],
        out_shape=[
            jax.ShapeDtypeStruct((_N, _D), jnp.float32),
            jax.ShapeDtypeStruct((2, _D), jnp.float32),
        ],
    )(h, ws, aggp, degp, b)


def _post_lsm_body(h_ref, ws_ref, agg_ref, degp_ref, b_ref, out_ref):
    pre = _pre_from_parts(h_ref, ws_ref, agg_ref, degp_ref, b_ref)
    mx = jnp.max(pre, axis=1, keepdims=True)
    ex = jnp.exp(pre - mx)
    lse = jnp.log(jnp.sum(ex, axis=1, keepdims=True)) + mx
    out_ref[...] = pre - lse


def _post_lsm(h, ws, aggp, degp, b):
    return pl.pallas_call(
        _post_lsm_body,
        grid=(_GRID,),
        in_specs=[
            pl.BlockSpec((_RB, _D), lambda i: (i, 0)),
            pl.BlockSpec((_D, _D), lambda i: (0, 0)),
            pl.BlockSpec((2, _RB, _D), lambda i: (0, i, 0)),
            pl.BlockSpec((2, _RB, 16), lambda i: (0, i, 0)),
            pl.BlockSpec((1, _D), lambda i: (0, 0)),
        ],
        out_specs=pl.BlockSpec((_RB, _D), lambda i: (i, 0)),
        out_shape=jax.ShapeDtypeStruct((_N, _D), jnp.float32),
    )(h, ws, aggp, degp, b)


def _sc_agg_call(m, srcm, dstm, zeros_nd):
    mesh = plsc.VectorSubcoreMesh(core_axis_name="c", subcore_axis_name="s")

    out_type = [jax.ShapeDtypeStruct((_NC, _N, _D), jnp.float32)]
    scratch = [
        pltpu.VMEM((_NCH, _C), jnp.int32),
        pltpu.VMEM((_NCH, _C), jnp.int32),
        [pltpu.VMEM((_C, _DH), jnp.float32) for _ in range(_NBUF)],
        [pltpu.SemaphoreType.DMA for _ in range(_NBUF)],
        [pltpu.SemaphoreType.DMA for _ in range(_NBUF)],
        pltpu.VMEM_SHARED((_N, _DH), jnp.float32),
    ]

    def body(m_hbm, srcm_hbm, dstm_hbm, z_nd_hbm, out_agg,
             src_v, dst_v, bufs, gsems, ssems, agg_sh):
        cid = lax.axis_index("c")
        sid = lax.axis_index("s")
        wid = cid * _NS + sid
        row0 = sid * _RPZ

        pltpu.sync_copy(srcm_hbm.at[pl.ds(wid * _NCH, _NCH)], src_v)
        pltpu.sync_copy(dstm_hbm.at[pl.ds(wid * _NCH, _NCH)], dst_v)

        for half in range(2):
            m_half = m_hbm.at[half]

            @pl.when(sid < _ZW)
            def _():
                pltpu.sync_copy(z_nd_hbm.at[pl.ds(row0, _RPZ)],
                                agg_sh.at[pl.ds(row0, _RPZ)])

            plsc.subcore_barrier()

            def gather(g, b):
                pltpu.async_copy(m_half.at[src_v.at[g]], bufs[b], gsems[b])

            def gwait(b):
                pltpu.make_async_copy(m_half.at[src_v.at[0]], bufs[b],
                                      gsems[b]).wait()

            def scat(g, b):
                pltpu.async_copy(bufs[b], agg_sh.at[dst_v.at[g]],
                                 ssems[b], add=True)

            def swait(b):
                pltpu.make_async_copy(bufs[b], agg_sh.at[dst_v.at[0]],
                                      ssems[b]).wait()

            for g in range(_PF):
                gather(g, g)
            for g in range(_PF):
                gwait(g)
                scat(g, g)
                gather(g + _PF, g + _PF)

            def super_iter(i, carry):
                for j in range(_NBUF):
                    g = _NBUF * i + _PF + j
                    b = (_PF + j) % _NBUF
                    gwait(b)
                    scat(g, b)
                    nb = (b + _PF) % _NBUF
                    swait(nb)
                    gather(g + _PF, nb)
                return carry

            lax.fori_loop(0, (_NCH - 2 * _PF) // _NBUF, super_iter, 0)

            for g in range(_NCH - _PF, _NCH):
                b = g % _NBUF
                gwait(b)
                scat(g, b)
                swait((b + _PF) % _NBUF)
            for g in range(_NCH - _PF, _NCH):
                swait(g % _NBUF)

            plsc.subcore_barrier()

            @pl.when(sid < _ZW)
            def _():
                pltpu.sync_copy(
                    agg_sh.at[pl.ds(row0, _RPZ)],
                    out_agg.at[cid, pl.ds(row0, _RPZ),
                               pl.ds(half * _DH, _DH)])

    return pl.kernel(body, out_type=out_type, mesh=mesh,
                     scratch_types=scratch,
                     compiler_params=pltpu.CompilerParams(
                         use_tc_tiling_on_sc=False))(
        m, srcm, dstm, zeros_nd)[0]


def _sc_deg_call(dstm, zeros16, ones16):
    mesh = plsc.VectorSubcoreMesh(core_axis_name="c", subcore_axis_name="s")
    out_type = [jax.ShapeDtypeStruct((_NC, _N, 16), jnp.float32)]
    scratch = [
        pltpu.VMEM((_NCH, _C), jnp.int32),
        pltpu.VMEM((_C, 16), jnp.float32),
        pltpu.VMEM_SHARED((_N, 16), jnp.float32),
        pltpu.SemaphoreType.DMA,
        pltpu.SemaphoreType.DMA,
    ]

    def body(dstm_hbm, z16_hbm, ones_hbm, out_deg,
             dst_v, ones_v, deg_sh, sem_a, sem_b):
        cid = lax.axis_index("c")
        sid = lax.axis_index("s")
        wid = cid * _NS + sid
        row0 = sid * _RPZ

        @pl.when(sid < _ZW)
        def _():
            pltpu.sync_copy(z16_hbm.at[pl.ds(row0, _RPZ)],
                            deg_sh.at[pl.ds(row0, _RPZ)])

        pltpu.sync_copy(ones_hbm, ones_v)
        pltpu.sync_copy(dstm_hbm.at[pl.ds(wid * _NCH, _NCH)], dst_v)
        plsc.subcore_barrier()

        def scat(g, sem):
            pltpu.async_copy(ones_v, deg_sh.at[dst_v.at[g]], sem, add=True)

        def swait(sem):
            pltpu.make_async_copy(ones_v, deg_sh.at[dst_v.at[0]], sem).wait()

        scat(0, sem_a)
        scat(1, sem_b)

        def step(i, carry):
            g = 2 * i

            @pl.when(g + 2 < _NCH)
            def _():
                scat(g + 2, sem_a)
                scat(g + 3, sem_b)

            swait(sem_a)
            swait(sem_b)
            return carry

        lax.fori_loop(0, _NCH // 2, step, 0)
        plsc.subcore_barrier()

        @pl.when(sid < _ZW)
        def _():
            pltpu.sync_copy(deg_sh.at[pl.ds(row0, _RPZ)],
                            out_deg.at[cid, pl.ds(row0, _RPZ)])

    return pl.kernel(body, out_type=out_type, mesh=mesh,
                     scratch_types=scratch,
                     compiler_params=pltpu.CompilerParams(
                         use_tc_tiling_on_sc=False))(
        dstm, zeros16, ones16)[0]


def kernel(x, edge_index, W_self0, W_neigh0, b0, gamma0, beta0,
           W_self1, W_neigh1, b1, gamma1, beta1, W_self2, W_neigh2, b2):
    srcm = edge_index[0].reshape(_E // _C, _C)
    dstm = edge_index[1].reshape(_E // _C, _C)
    zeros_nd = jnp.zeros((_N, _DH), jnp.float32)
    zeros16 = jnp.zeros((_N, 16), jnp.float32)
    ones16 = jnp.ones((_C, 16), jnp.float32)
    b0r, b1r, b2r = (b.reshape(1, _D) for b in (b0, b1, b2))
    g0r, g1r = gamma0.reshape(1, _D), gamma1.reshape(1, _D)
    be0r, be1r = beta0.reshape(1, _D), beta1.reshape(1, _D)

    degp = _sc_deg_call(dstm, zeros16, ones16)
    m0 = _mm_m(x, W_neigh0)
    aggp0 = _sc_agg_call(m0, srcm, dstm, zeros_nd)
    m1, h1 = _post_bnr(x, W_self0, aggp0, degp, b0r, g0r, be0r, W_neigh1)
    aggp1 = _sc_agg_call(m1, srcm, dstm, zeros_nd)
    m2, h2 = _post_bnr(h1, W_self1, aggp1, degp, b1r, g1r, be1r, W_neigh2)
    aggp2 = _sc_agg_call(m2, srcm, dstm, zeros_nd)
    return _post_lsm(h2, W_self2, aggp2, degp, b2r)
